# baseline (speedup 1.0000x reference)
import jax
import jax.numpy as jnp
from jax.experimental import pallas as pl
from jax.experimental.pallas import tpu as pltpu

EPS = 1e-5
VMEM_LIMIT_BYTES = 48 * 1024 * 1024


def _fill_pad(pad_ref, interior, H, W, C):
    Hp, Wp = H + 2, W + 2
    pad_ref[0:1, :, :] = jnp.zeros((1, Wp, C), jnp.bfloat16)
    pad_ref[H + 1:H + 2, :, :] = jnp.zeros((1, Wp, C), jnp.bfloat16)
    pad_ref[:, 0:1, :] = jnp.zeros((Hp, 1, C), jnp.bfloat16)
    pad_ref[:, W + 1:W + 2, :] = jnp.zeros((Hp, 1, C), jnp.bfloat16)
    pad_ref[1:H + 1, 1:W + 1, :] = interior


def _conv_stats(pad_ref, patch_ref, w_ref, stat_ref, H, W, C):
    apad = pad_ref[...]
    for j in range(9):
        kh, kw = divmod(j, 3)
        patch_ref[:, j * C:(j + 1) * C] = (
            apad[kh:kh + H, kw:kw + W, :].reshape(H * W, C))
    y = jnp.dot(patch_ref[...], w_ref[...],
                preferred_element_type=jnp.float32)
    Co = y.shape[1]
    stat_ref[0:1, 0:1, :] = jnp.sum(y, axis=0).reshape(1, 1, Co)
    stat_ref[0:1, 1:2, :] = jnp.sum(y * y, axis=0).reshape(1, 1, Co)
    return y


def _conv1_kernel(x_ref, w_ref, y_ref, stat_ref, pad_ref, patch_ref):
    _, H, W, C = x_ref.shape
    _fill_pad(pad_ref, x_ref[...].reshape(H, W, C), H, W, C)
    y = _conv_stats(pad_ref, patch_ref, w_ref, stat_ref, H, W, C)
    y_ref[...] = y.astype(jnp.bfloat16).reshape(1, H, W, y.shape[1])


def _conv2_kernel(y1_ref, s_ref, t_ref, w_ref, y_ref, stat_ref,
                  pad_ref, patch_ref):
    _, H, W, C = y1_ref.shape
    a = (y1_ref[...].reshape(H, W, C).astype(jnp.float32) * s_ref[...]
         + t_ref[...])
    a = jnp.maximum(a, 0.0).astype(jnp.bfloat16)
    _fill_pad(pad_ref, a, H, W, C)
    y = _conv_stats(pad_ref, patch_ref, w_ref, stat_ref, H, W, C)
    y_ref[...] = y.astype(jnp.bfloat16).reshape(1, H, W, y.shape[1])


def _epilogue_kernel(y2_ref, x_ref, s_ref, t_ref, o_ref):
    y = (y2_ref[...].astype(jnp.float32) * s_ref[...] + t_ref[...]
         + x_ref[...].astype(jnp.float32))
    o_ref[...] = jnp.maximum(y, 0.0)


def _finalize_bn(stat_partials, gamma, beta, count):
    s = jnp.sum(stat_partials, axis=0)
    mean = s[0] / count
    var = jnp.maximum(s[1] / count - mean * mean, 0.0)
    inv = jax.lax.rsqrt(var + EPS)
    scale = gamma * inv
    shift = beta - mean * scale
    C = scale.shape[0]
    return scale.reshape(1, C), shift.reshape(1, C)


@jax.jit
def _basic_block(x_nchw, w1, g1, b1, w2, g2, b2):
    N, Cin, H, W = x_nchw.shape
    Cout = w1.shape[-1]

    x = jnp.transpose(x_nchw, (0, 2, 3, 1)).astype(jnp.bfloat16)
    w1m = w1.reshape(9 * Cin, Cout).astype(jnp.bfloat16)
    w2m = w2.reshape(9 * Cout, Cout).astype(jnp.bfloat16)
    count = float(N * H * W)

    cparams = pltpu.CompilerParams(
        dimension_semantics=("parallel",),
        vmem_limit_bytes=VMEM_LIMIT_BYTES)

    def act_spec(C):
        return pl.BlockSpec((1, H, W, C), lambda n: (n, 0, 0, 0))

    def resident_spec(shape):
        return pl.BlockSpec(shape, lambda n: (0,) * len(shape))

    stat_spec = pl.BlockSpec((1, 2, Cout), lambda n: (n, 0, 0))

    y1, stat1 = pl.pallas_call(
        _conv1_kernel,
        grid=(N,),
        in_specs=[act_spec(Cin), resident_spec((9 * Cin, Cout))],
        out_specs=(act_spec(Cout), stat_spec),
        out_shape=(jax.ShapeDtypeStruct((N, H, W, Cout), jnp.bfloat16),
                   jax.ShapeDtypeStruct((N, 2, Cout), jnp.float32)),
        scratch_shapes=[pltpu.VMEM((H + 2, W + 2, Cin), jnp.bfloat16),
                        pltpu.VMEM((H * W, 9 * Cin), jnp.bfloat16)],
        compiler_params=cparams,
    )(x, w1m)

    scale1, shift1 = _finalize_bn(stat1, g1, b1, count)

    y2, stat2 = pl.pallas_call(
        _conv2_kernel,
        grid=(N,),
        in_specs=[act_spec(Cout), resident_spec((1, Cout)),
                  resident_spec((1, Cout)), resident_spec((9 * Cout, Cout))],
        out_specs=(act_spec(Cout), stat_spec),
        out_shape=(jax.ShapeDtypeStruct((N, H, W, Cout), jnp.bfloat16),
                   jax.ShapeDtypeStruct((N, 2, Cout), jnp.float32)),
        scratch_shapes=[pltpu.VMEM((H + 2, W + 2, Cout), jnp.bfloat16),
                        pltpu.VMEM((H * W, 9 * Cout), jnp.bfloat16)],
        compiler_params=cparams,
    )(y1, scale1, shift1, w2m)

    scale2, shift2 = _finalize_bn(stat2, g2, b2, count)

    out_nhwc = pl.pallas_call(
        _epilogue_kernel,
        grid=(N,),
        in_specs=[act_spec(Cout), act_spec(Cin),
                  resident_spec((1, Cout)), resident_spec((1, Cout))],
        out_specs=act_spec(Cout),
        out_shape=jax.ShapeDtypeStruct((N, H, W, Cout), jnp.float32),
        compiler_params=cparams,
    )(y2, x, scale2, shift2)

    return jnp.transpose(out_nhwc, (0, 3, 1, 2))


def kernel(x_nchw, w1, g1, b1, w2, g2, b2):
    return _basic_block(x_nchw, w1, g1, b1, w2, g2, b2)
